# baseline (speedup 1.0000x reference)
import functools

import jax
import jax.numpy as jnp
from jax import lax
from jax.experimental import pallas as pl
from jax.experimental.pallas import tpu as pltpu
from jax.experimental.pallas import tpu_sc as plsc

B, S, H = 128, 512, 768
L = 16
HV = H // L
NC, NS = 2, 16
NW = NC * NS
ROWS_PER_W = B // NW
CK = 32
NCHUNK = S // CK
POS_PER_SUB = S // NS
EPS = 1e-12


def _lanesum(v):
    v = v + lax.rev(v, (0,))
    s = v[0]
    for i in range(1, 8):
        s = s + v[i]
    return s


def _rsqrt(x):
    i = lax.bitcast_convert_type(x, jnp.int32)
    i = jnp.int32(0x5F3759DF) - lax.shift_right_logical(i, 1)
    y = lax.bitcast_convert_type(i, jnp.float32)
    half = x * jnp.float32(0.5)
    for _ in range(4):
        y = y * (jnp.float32(1.5) - half * y * y)
    return y


def _body(ids_hbm, tt_hbm, wemb_hbm, pemb_hbm, temb_hbm, gb_hbm, out_hbm,
          idx_v, ttv, wbuf, obuf, pbuf, tbuf, gbuf, pshared, sem, psem):
    cid = lax.axis_index("c")
    sid = lax.axis_index("s")
    wid = sid * NC + cid

    pltpu.sync_copy(temb_hbm, tbuf)
    pltpu.sync_copy(gb_hbm, gbuf)

    p0 = sid * POS_PER_SUB
    pltpu.async_copy(pemb_hbm.at[pl.ds(p0, POS_PER_SUB)], pbuf, psem).wait()
    pltpu.async_copy(pbuf, pshared.at[pl.ds(p0, POS_PER_SUB)], psem).wait()
    plsc.subcore_barrier()

    def chunk_body(c, _):
        s0 = c * CK
        for r in range(ROWS_PER_W):
            row = wid * ROWS_PER_W + r
            pltpu.sync_copy(ids_hbm.at[row, pl.ds(s0, CK)], idx_v)
            pltpu.sync_copy(tt_hbm.at[row, pl.ds(s0, CK)],
                            ttv.at[pl.ds(0, CK)])
            pltpu.async_copy(pshared.at[pl.ds(s0, CK)], pbuf, psem).wait()
            pltpu.async_copy(wemb_hbm.at[idx_v], wbuf, sem).wait()

            @plsc.parallel_loop(0, CK, step=1, unroll=2)
            def _(k):
                ttk = ttv[pl.ds(k, L)][0]
                sa = [jnp.zeros((L,), jnp.float32) for _ in range(4)]
                qa = [jnp.zeros((L,), jnp.float32) for _ in range(4)]
                for j in range(HV):
                    v = (wbuf[k, pl.ds(j * L, L)] + pbuf[k, pl.ds(j * L, L)]
                         ) + tbuf[ttk, pl.ds(j * L, L)]
                    sa[j % 4] = sa[j % 4] + v
                    qa[j % 4] = qa[j % 4] + v * v
                sumv = (sa[0] + sa[1]) + (sa[2] + sa[3])
                sqv = (qa[0] + qa[1]) + (qa[2] + qa[3])
                mean = _lanesum(sumv) * (1.0 / H)
                var = _lanesum(sqv) * (1.0 / H) - mean * mean
                rstd = _rsqrt(var + EPS)
                mean_v = jnp.full((L,), mean, jnp.float32)
                rstd_v = jnp.full((L,), rstd, jnp.float32)
                for j in range(HV):
                    v = (wbuf[k, pl.ds(j * L, L)] + pbuf[k, pl.ds(j * L, L)]
                         ) + tbuf[ttk, pl.ds(j * L, L)]
                    g = gbuf[0, pl.ds(j * L, L)]
                    b = gbuf[1, pl.ds(j * L, L)]
                    obuf[k, pl.ds(j * L, L)] = (v - mean_v) * (rstd_v * g) + b

            pltpu.sync_copy(obuf, out_hbm.at[row, pl.ds(s0, CK)])
        return 0

    lax.fori_loop(0, NCHUNK, chunk_body, 0)


_mesh = plsc.VectorSubcoreMesh(core_axis_name="c", subcore_axis_name="s",
                               num_cores=NC, num_subcores=NS)

_emb_ln = pl.kernel(
    _body,
    out_type=jax.ShapeDtypeStruct((B, S, H), jnp.float32),
    mesh=_mesh,
    scratch_types=[
        pltpu.VMEM((CK,), jnp.int32),
        pltpu.VMEM((CK + L,), jnp.int32),
        pltpu.VMEM((CK, H), jnp.float32),
        pltpu.VMEM((CK, H), jnp.float32),
        pltpu.VMEM((CK, H), jnp.float32),
        pltpu.VMEM((2, H), jnp.float32),
        pltpu.VMEM((2, H), jnp.float32),
        pltpu.VMEM_SHARED((S, H), jnp.float32),
        pltpu.SemaphoreType.DMA,
        pltpu.SemaphoreType.DMA,
    ],
)


@jax.jit
def kernel(input_ids, token_type_ids, word_emb, pos_emb, type_emb, gamma, beta):
    ids = input_ids.astype(jnp.int32)
    tts = token_type_ids.astype(jnp.int32)
    gb = jnp.stack([gamma, beta])
    return _emb_ln(ids, tts, word_emb, pos_emb, type_emb, gb)


# speedup vs baseline: 4.0660x; 4.0660x over previous
import functools

import jax
import jax.numpy as jnp
from jax import lax
from jax.experimental import pallas as pl
from jax.experimental.pallas import tpu as pltpu
from jax.experimental.pallas import tpu_sc as plsc

VOCAB, B, S, H = 30522, 128, 512, 768
NC, NS = 2, 16
NW = NC * NS
TOK_PER_W = B * S // NW
GK = 64
NG = TOK_PER_W // GK
EPS = 1e-12


def _gather_body(ids_hbm, wemb_hbm, out_hbm,
                 idx0, idx1, buf0, buf1, gs0, gs1, os0, os1):
    cid = lax.axis_index("c")
    sid = lax.axis_index("s")
    wid = sid * NC + cid
    base = wid * TOK_PER_W

    idx = (idx0, idx1)
    buf = (buf0, buf1)
    gsem = (gs0, gs1)
    osem = (os0, os1)

    def start_gather(i, b):
        pltpu.sync_copy(ids_hbm.at[pl.ds(base + i * GK, GK)], idx[b])
        return pltpu.async_copy(wemb_hbm.at[idx[b]], buf[b], gsem[b])

    gathers = [None, None]
    outs = [None, None]
    gathers[0] = start_gather(0, 0)
    for i in range(NG):
        b = i % 2
        gathers[b].wait()
        if i + 1 < NG:
            if outs[1 - b] is not None:
                outs[1 - b].wait()
            gathers[1 - b] = start_gather(i + 1, 1 - b)
        outs[b] = pltpu.async_copy(
            buf[b], out_hbm.at[pl.ds(base + i * GK, GK)], osem[b])
    outs[(NG - 1) % 2].wait()
    outs[NG % 2].wait()


_mesh = plsc.VectorSubcoreMesh(core_axis_name="c", subcore_axis_name="s",
                               num_cores=NC, num_subcores=NS)

_sc_gather = pl.kernel(
    _gather_body,
    out_type=jax.ShapeDtypeStruct((B * S, H), jnp.float32),
    mesh=_mesh,
    scratch_types=[
        pltpu.VMEM((GK,), jnp.int32),
        pltpu.VMEM((GK,), jnp.int32),
        pltpu.VMEM((GK, H), jnp.float32),
        pltpu.VMEM((GK, H), jnp.float32),
        pltpu.SemaphoreType.DMA,
        pltpu.SemaphoreType.DMA,
        pltpu.SemaphoreType.DMA,
        pltpu.SemaphoreType.DMA,
    ],
)


def _ln_body(w_ref, tt_ref, pos_ref, type_ref, gamma_ref, beta_ref, out_ref):
    w = w_ref[0]
    tt = tt_ref[0]
    pos = pos_ref[...]
    tsel = jnp.where((tt[0][:, None]) == 1, type_ref[1][None, :],
                     type_ref[0][None, :])
    v = w + pos + tsel
    mean = jnp.mean(v, axis=-1, keepdims=True)
    c = v - mean
    var = jnp.mean(c * c, axis=-1, keepdims=True)
    normed = c * jax.lax.rsqrt(var + EPS)
    out_ref[0] = normed * gamma_ref[...][None, :] + beta_ref[...][None, :]


_tc_ln = pl.pallas_call(
    _ln_body,
    grid=(B,),
    in_specs=[
        pl.BlockSpec((1, S, H), lambda i: (i, 0, 0)),
        pl.BlockSpec((1, 1, S), lambda i: (i, 0, 0)),
        pl.BlockSpec((S, H), lambda i: (0, 0)),
        pl.BlockSpec((2, H), lambda i: (0, 0)),
        pl.BlockSpec((H,), lambda i: (0,)),
        pl.BlockSpec((H,), lambda i: (0,)),
    ],
    out_specs=pl.BlockSpec((1, S, H), lambda i: (i, 0, 0)),
    out_shape=jax.ShapeDtypeStruct((B, S, H), jnp.float32),
)


@jax.jit
def kernel(input_ids, token_type_ids, word_emb, pos_emb, type_emb, gamma, beta):
    ids = input_ids.astype(jnp.int32).reshape(B * S)
    tts = token_type_ids.astype(jnp.int32).reshape(B, 1, S)
    gathered = _sc_gather(ids, word_emb).reshape(B, S, H)
    return _tc_ln(gathered, tts, pos_emb, type_emb, gamma, beta)
